# baseline (speedup 1.0000x reference)
import functools

import jax
import jax.numpy as jnp
from jax import lax
from jax.experimental import pallas as pl
from jax.experimental.pallas import tpu as pltpu
from jax.experimental.pallas import tpu_sc as plsc

BINS = 10
BLOCK_ROWS = 1024

SC_ROWS = 4096
SC_NW = 32
SC_CHUNK = 8192
SC_NTH = BINS - 1


def _sc_hist(x_flat, t_flat, *, base, n_elems):
    e_w = n_elems // SC_NW
    n_chunks = e_w // SC_CHUNK
    mesh = plsc.VectorSubcoreMesh(core_axis_name="c", subcore_axis_name="s")

    @functools.partial(
        pl.kernel,
        out_type=jax.ShapeDtypeStruct((SC_NW * SC_NTH, 16), jnp.float32),
        mesh=mesh,
        scratch_types=[
            pltpu.VMEM((SC_CHUNK,), jnp.float32),
            pltpu.VMEM((SC_CHUNK,), jnp.float32),
        ]
        + [pltpu.VMEM((16,), jnp.float32) for _ in range(SC_NTH)],
    )
    def k(x_hbm, t_hbm, out_hbm, xbuf, tbuf, *gebufs):
        wid = lax.axis_index("s") * 2 + lax.axis_index("c")
        wbase = base + wid * e_w

        def chunk_body(ci, accs):
            off = wbase + ci * SC_CHUNK
            pltpu.sync_copy(x_hbm.at[pl.ds(off, SC_CHUNK)], xbuf)
            pltpu.sync_copy(t_hbm.at[pl.ds(off, SC_CHUNK)], tbuf)

            def vec_body(vi, a):
                xv = xbuf[pl.ds(vi * 16, 16)]
                tv = tbuf[pl.ds(vi * 16, 16)]
                e = jnp.exp(-jnp.abs(xv))
                p = jnp.where(xv >= 0.0, 1.0, e) / (1.0 + e)
                g10 = jnp.abs(p - tv) * BINS
                return tuple(
                    acc + jnp.where(g10 >= (j + 1), 1.0, 0.0)
                    for j, acc in enumerate(a)
                )

            return lax.fori_loop(0, SC_CHUNK // 16, vec_body, accs, unroll=8)

        zero = jnp.zeros((16,), jnp.float32)
        accs = lax.fori_loop(0, n_chunks, chunk_body, (zero,) * SC_NTH)
        for j in range(SC_NTH):
            gebufs[j][...] = accs[j]
            pltpu.sync_copy(gebufs[j], out_hbm.at[wid * SC_NTH + j])

    return k(x_flat, t_flat)


def _g10(x, t):
    h = jnp.tanh(jnp.abs(x) * 0.5)
    d = 0.5 + 0.5 * h
    p = jnp.where(x >= 0.0, d, 1.0 - d)
    return jnp.abs(p - t) * BINS, d


def _hist_kernel(x_ref, t_ref, cnt_ref):
    i = pl.program_id(0)

    @pl.when(i == 0)
    def _init():
        for k in range(BINS):
            cnt_ref[0, k] = 0

    g10, _ = _g10(x_ref[...], t_ref[...])
    b = jnp.minimum(g10.astype(jnp.int32), BINS - 1)
    islo = b < 5
    sh6 = b * 6
    sh = jnp.where(islo, sh6, sh6 - 30)
    p = jnp.left_shift(jnp.int32(1), sh)
    plo = jnp.where(islo, p, 0)
    phi = p - plo
    for arr, base in ((plo, 0), (phi, 5)):
        s = arr
        for _ in range(5):
            h = s.shape[0] // 2
            s = s[:h] + s[h:]
        for f in range(5):
            cnt_ref[0, base + f] += jnp.sum((s >> (6 * f)) & 63)


def _loss_kernel(cnt_ref, x_ref, t_ref, out_ref, *, tot):
    del tot
    counts = [cnt_ref[0, k] for k in range(BINS)]
    n = functools.reduce(
        lambda a, b: a + b, [(c > 0).astype(jnp.int32) for c in counts]
    )
    nf = n.astype(jnp.float32)
    c = [1.0 / (jnp.maximum(cn, 1).astype(jnp.float32) * nf) for cn in counts]

    x = x_ref[...]
    t = t_ref[...]
    g10, d = _g10(x, t)
    w_lo = jnp.where(
        g10 >= 2.0,
        jnp.where(g10 >= 3.0, jnp.where(g10 >= 4.0, c[4], c[3]), c[2]),
        jnp.where(g10 >= 1.0, c[1], c[0]),
    )
    w_hi = jnp.where(
        g10 >= 7.0,
        jnp.where(g10 >= 8.0, jnp.where(g10 >= 9.0, c[9], c[8]), c[7]),
        jnp.where(g10 >= 6.0, c[6], c[5]),
    )
    w = jnp.where(g10 >= 5.0, w_hi, w_lo)
    bce = jnp.maximum(x, 0.0) - jnp.log(d) - x * t
    out_ref[...] = w * bce


def kernel(inputs, targets):
    rows, cols = inputs.shape
    tot = rows * cols
    tc_rows = rows - SC_ROWS
    sc_elems = SC_ROWS * cols
    data_spec = pl.BlockSpec((BLOCK_ROWS, cols), lambda i: (i, 0))

    sc_part = _sc_hist(
        inputs.reshape(-1), targets.reshape(-1), base=tc_rows * cols,
        n_elems=sc_elems,
    )

    tc_cnt = pl.pallas_call(
        _hist_kernel,
        grid=(tc_rows // BLOCK_ROWS,),
        in_specs=[data_spec, data_spec],
        out_specs=pl.BlockSpec(memory_space=pltpu.SMEM),
        out_shape=jax.ShapeDtypeStruct((1, BINS), jnp.int32),
        compiler_params=pltpu.CompilerParams(
            dimension_semantics=("arbitrary",),
        ),
    )(inputs, targets)

    sc_ge = (
        sc_part.reshape(SC_NW, SC_NTH, 16).sum(axis=(0, 2)).astype(jnp.int32)
    )
    ge_hi = jnp.concatenate([sc_ge, jnp.zeros((1,), jnp.int32)])
    ge_lo = jnp.concatenate([jnp.full((1,), sc_elems, jnp.int32), sc_ge])
    cnt = tc_cnt + (ge_lo - ge_hi)[None, :]

    loss = pl.pallas_call(
        functools.partial(_loss_kernel, tot=tot),
        grid=(rows // BLOCK_ROWS,),
        in_specs=[
            pl.BlockSpec(memory_space=pltpu.SMEM),
            data_spec,
            data_spec,
        ],
        out_specs=data_spec,
        out_shape=jax.ShapeDtypeStruct((rows, cols), jnp.float32),
        compiler_params=pltpu.CompilerParams(
            dimension_semantics=("parallel",),
        ),
    )(cnt, inputs, targets)
    return loss


# speedup vs baseline: 1.3127x; 1.3127x over previous
import functools

import jax
import jax.numpy as jnp
from jax import lax
from jax.experimental import pallas as pl
from jax.experimental.pallas import tpu as pltpu
from jax.experimental.pallas import tpu_sc as plsc

BINS = 10
BLOCK_ROWS = 1024

SC_ROWS = 4096
SC_NW = 32
SC_CHUNK = 8192
SC_NTH = BINS - 1


def _sc_hist(x2d, t2d, *, base_row, n_rows):
    rows_per_w = n_rows // SC_NW
    cr = 8
    cols = x2d.shape[1]
    mesh = plsc.VectorSubcoreMesh(core_axis_name="c", subcore_axis_name="s")

    @functools.partial(
        pl.kernel,
        out_type=jax.ShapeDtypeStruct((SC_NW * SC_NTH, 16), jnp.float32),
        mesh=mesh,
        scratch_types=[
            pltpu.VMEM((cr, cols), jnp.float32),
            pltpu.VMEM((cr, cols), jnp.float32),
        ]
        + [pltpu.VMEM((16,), jnp.float32) for _ in range(SC_NTH)],
    )
    def k(x_hbm, t_hbm, out_hbm, xbuf, tbuf, *gebufs):
        wid = lax.axis_index("s") * 2 + lax.axis_index("c")
        wbase = base_row + wid * rows_per_w

        def chunk_body(ci, accs):
            r0 = wbase + ci * cr
            pltpu.sync_copy(x_hbm.at[pl.ds(r0, cr)], xbuf)
            pltpu.sync_copy(t_hbm.at[pl.ds(r0, cr)], tbuf)

            def row_body(r, a1):
                def vec_body(vi, a):
                    xv = xbuf[r, pl.ds(vi * 16, 16)]
                    tv = tbuf[r, pl.ds(vi * 16, 16)]
                    e = jnp.exp(-jnp.abs(xv))
                    p = jnp.where(xv >= 0.0, 1.0, e) / (1.0 + e)
                    g10 = jnp.abs(p - tv) * BINS
                    return tuple(
                        acc + jnp.where(g10 >= (j + 1), 1.0, 0.0)
                        for j, acc in enumerate(a)
                    )

                return lax.fori_loop(0, cols // 16, vec_body, a1, unroll=8)

            return lax.fori_loop(0, cr, row_body, accs)

        zero = jnp.zeros((16,), jnp.float32)
        accs = lax.fori_loop(0, rows_per_w // cr, chunk_body, (zero,) * SC_NTH)
        for j in range(SC_NTH):
            gebufs[j][...] = accs[j]
            pltpu.sync_copy(gebufs[j], out_hbm.at[wid * SC_NTH + j])

    return k(x2d, t2d)


def _g10(x, t):
    h = jnp.tanh(jnp.abs(x) * 0.5)
    d = 0.5 + 0.5 * h
    p = jnp.where(x >= 0.0, d, 1.0 - d)
    return jnp.abs(p - t) * BINS, d


def _hist_kernel(x_ref, t_ref, cnt_ref):
    i = pl.program_id(0)

    @pl.when(i == 0)
    def _init():
        for k in range(BINS):
            cnt_ref[0, k] = 0

    g10, _ = _g10(x_ref[...], t_ref[...])
    b = jnp.minimum(g10.astype(jnp.int32), BINS - 1)
    islo = b < 5
    sh6 = b * 6
    sh = jnp.where(islo, sh6, sh6 - 30)
    p = jnp.left_shift(jnp.int32(1), sh)
    plo = jnp.where(islo, p, 0)
    phi = p - plo
    for arr, base in ((plo, 0), (phi, 5)):
        s = arr
        for _ in range(5):
            h = s.shape[0] // 2
            s = s[:h] + s[h:]
        for f in range(5):
            cnt_ref[0, base + f] += jnp.sum((s >> (6 * f)) & 63)


def _loss_kernel(cnt_ref, x_ref, t_ref, out_ref, *, tot):
    del tot
    counts = [cnt_ref[0, k] for k in range(BINS)]
    n = functools.reduce(
        lambda a, b: a + b, [(c > 0).astype(jnp.int32) for c in counts]
    )
    nf = n.astype(jnp.float32)
    c = [1.0 / (jnp.maximum(cn, 1).astype(jnp.float32) * nf) for cn in counts]

    x = x_ref[...]
    t = t_ref[...]
    g10, d = _g10(x, t)
    w_lo = jnp.where(
        g10 >= 2.0,
        jnp.where(g10 >= 3.0, jnp.where(g10 >= 4.0, c[4], c[3]), c[2]),
        jnp.where(g10 >= 1.0, c[1], c[0]),
    )
    w_hi = jnp.where(
        g10 >= 7.0,
        jnp.where(g10 >= 8.0, jnp.where(g10 >= 9.0, c[9], c[8]), c[7]),
        jnp.where(g10 >= 6.0, c[6], c[5]),
    )
    w = jnp.where(g10 >= 5.0, w_hi, w_lo)
    bce = jnp.maximum(x, 0.0) - jnp.log(d) - x * t
    out_ref[...] = w * bce


def kernel(inputs, targets):
    rows, cols = inputs.shape
    tot = rows * cols
    tc_rows = rows - SC_ROWS
    sc_elems = SC_ROWS * cols
    data_spec = pl.BlockSpec((BLOCK_ROWS, cols), lambda i: (i, 0))

    sc_part = _sc_hist(inputs, targets, base_row=tc_rows, n_rows=SC_ROWS)

    tc_cnt = pl.pallas_call(
        _hist_kernel,
        grid=(tc_rows // BLOCK_ROWS,),
        in_specs=[data_spec, data_spec],
        out_specs=pl.BlockSpec(memory_space=pltpu.SMEM),
        out_shape=jax.ShapeDtypeStruct((1, BINS), jnp.int32),
        compiler_params=pltpu.CompilerParams(
            dimension_semantics=("arbitrary",),
        ),
    )(inputs, targets)

    sc_ge = (
        sc_part.reshape(SC_NW, SC_NTH, 16).sum(axis=(0, 2)).astype(jnp.int32)
    )
    ge_hi = jnp.concatenate([sc_ge, jnp.zeros((1,), jnp.int32)])
    ge_lo = jnp.concatenate([jnp.full((1,), sc_elems, jnp.int32), sc_ge])
    cnt = tc_cnt + (ge_lo - ge_hi)[None, :]

    loss = pl.pallas_call(
        functools.partial(_loss_kernel, tot=tot),
        grid=(rows // BLOCK_ROWS,),
        in_specs=[
            pl.BlockSpec(memory_space=pltpu.SMEM),
            data_spec,
            data_spec,
        ],
        out_specs=data_spec,
        out_shape=jax.ShapeDtypeStruct((rows, cols), jnp.float32),
        compiler_params=pltpu.CompilerParams(
            dimension_semantics=("parallel",),
        ),
    )(cnt, inputs, targets)
    return loss


# speedup vs baseline: 1.8930x; 1.4420x over previous
import functools

import jax
import jax.numpy as jnp
from jax import lax
from jax.experimental import pallas as pl
from jax.experimental.pallas import tpu as pltpu
from jax.experimental.pallas import tpu_sc as plsc

BINS = 10
BLOCK_ROWS = 1024

SC_ROWS = 1024
SC_NW = 32
SC_CHUNK = 8192
SC_NTH = BINS - 1


def _sc_hist(x2d, t2d, *, base_row, n_rows):
    rows_per_w = n_rows // SC_NW
    cr = 8
    cols = x2d.shape[1]
    mesh = plsc.VectorSubcoreMesh(core_axis_name="c", subcore_axis_name="s")

    @functools.partial(
        pl.kernel,
        out_type=jax.ShapeDtypeStruct((SC_NW * SC_NTH, 16), jnp.float32),
        mesh=mesh,
        scratch_types=[
            pltpu.VMEM((cr, cols), jnp.float32),
            pltpu.VMEM((cr, cols), jnp.float32),
        ]
        + [pltpu.VMEM((16,), jnp.float32) for _ in range(SC_NTH)],
    )
    def k(x_hbm, t_hbm, out_hbm, xbuf, tbuf, *gebufs):
        wid = lax.axis_index("s") * 2 + lax.axis_index("c")
        wbase = base_row + wid * rows_per_w

        def chunk_body(ci, accs):
            r0 = wbase + ci * cr
            pltpu.sync_copy(x_hbm.at[pl.ds(r0, cr)], xbuf)
            pltpu.sync_copy(t_hbm.at[pl.ds(r0, cr)], tbuf)

            def row_body(r, a1):
                def vec_body(vi, a):
                    xv = xbuf[r, pl.ds(vi * 16, 16)]
                    tv = tbuf[r, pl.ds(vi * 16, 16)]
                    e = jnp.exp(-jnp.abs(xv))
                    p = jnp.where(xv >= 0.0, 1.0, e) / (1.0 + e)
                    g10 = jnp.abs(p - tv) * BINS
                    return tuple(
                        acc + jnp.where(g10 >= (j + 1), 1.0, 0.0)
                        for j, acc in enumerate(a)
                    )

                return lax.fori_loop(0, cols // 16, vec_body, a1, unroll=8)

            return lax.fori_loop(0, cr, row_body, accs)

        zero = jnp.zeros((16,), jnp.float32)
        accs = lax.fori_loop(0, rows_per_w // cr, chunk_body, (zero,) * SC_NTH)
        for j in range(SC_NTH):
            gebufs[j][...] = accs[j]
            pltpu.sync_copy(gebufs[j], out_hbm.at[wid * SC_NTH + j])

    return k(x2d, t2d)


def _g10(x, t):
    h = jnp.tanh(jnp.abs(x) * 0.5)
    d = 0.5 + 0.5 * h
    p = jnp.where(x >= 0.0, d, 1.0 - d)
    return jnp.abs(p - t) * BINS, d


def _hist_kernel(x_ref, t_ref, cnt_ref):
    i = pl.program_id(0)

    @pl.when(i == 0)
    def _init():
        for k in range(BINS):
            cnt_ref[0, k] = 0

    g10, _ = _g10(x_ref[...], t_ref[...])
    b = jnp.minimum(g10.astype(jnp.int32), BINS - 1)
    islo = b < 5
    sh6 = b * 6
    sh = jnp.where(islo, sh6, sh6 - 30)
    p = jnp.left_shift(jnp.int32(1), sh)
    plo = jnp.where(islo, p, 0)
    phi = p - plo
    for arr, base in ((plo, 0), (phi, 5)):
        s = arr
        for _ in range(5):
            h = s.shape[0] // 2
            s = s[:h] + s[h:]
        for f in range(5):
            cnt_ref[0, base + f] += jnp.sum((s >> (6 * f)) & 63)


def _loss_kernel(cnt_ref, x_ref, t_ref, out_ref, *, tot):
    del tot
    counts = [cnt_ref[0, k] for k in range(BINS)]
    n = functools.reduce(
        lambda a, b: a + b, [(c > 0).astype(jnp.int32) for c in counts]
    )
    nf = n.astype(jnp.float32)
    c = [1.0 / (jnp.maximum(cn, 1).astype(jnp.float32) * nf) for cn in counts]

    x = x_ref[...]
    t = t_ref[...]
    g10, d = _g10(x, t)
    w_lo = jnp.where(
        g10 >= 2.0,
        jnp.where(g10 >= 3.0, jnp.where(g10 >= 4.0, c[4], c[3]), c[2]),
        jnp.where(g10 >= 1.0, c[1], c[0]),
    )
    w_hi = jnp.where(
        g10 >= 7.0,
        jnp.where(g10 >= 8.0, jnp.where(g10 >= 9.0, c[9], c[8]), c[7]),
        jnp.where(g10 >= 6.0, c[6], c[5]),
    )
    w = jnp.where(g10 >= 5.0, w_hi, w_lo)
    bce = jnp.maximum(x, 0.0) - jnp.log(d) - x * t
    out_ref[...] = w * bce


def kernel(inputs, targets):
    rows, cols = inputs.shape
    tot = rows * cols
    tc_rows = rows - SC_ROWS
    sc_elems = SC_ROWS * cols
    data_spec = pl.BlockSpec((BLOCK_ROWS, cols), lambda i: (i, 0))

    sc_part = _sc_hist(inputs, targets, base_row=tc_rows, n_rows=SC_ROWS)

    tc_cnt = pl.pallas_call(
        _hist_kernel,
        grid=(tc_rows // BLOCK_ROWS,),
        in_specs=[data_spec, data_spec],
        out_specs=pl.BlockSpec(memory_space=pltpu.SMEM),
        out_shape=jax.ShapeDtypeStruct((1, BINS), jnp.int32),
        compiler_params=pltpu.CompilerParams(
            dimension_semantics=("arbitrary",),
        ),
    )(inputs, targets)

    sc_ge = (
        sc_part.reshape(SC_NW, SC_NTH, 16).sum(axis=(0, 2)).astype(jnp.int32)
    )
    ge_hi = jnp.concatenate([sc_ge, jnp.zeros((1,), jnp.int32)])
    ge_lo = jnp.concatenate([jnp.full((1,), sc_elems, jnp.int32), sc_ge])
    cnt = tc_cnt + (ge_lo - ge_hi)[None, :]

    loss = pl.pallas_call(
        functools.partial(_loss_kernel, tot=tot),
        grid=(rows // BLOCK_ROWS,),
        in_specs=[
            pl.BlockSpec(memory_space=pltpu.SMEM),
            data_spec,
            data_spec,
        ],
        out_specs=data_spec,
        out_shape=jax.ShapeDtypeStruct((rows, cols), jnp.float32),
        compiler_params=pltpu.CompilerParams(
            dimension_semantics=("parallel",),
        ),
    )(cnt, inputs, targets)
    return loss


# speedup vs baseline: 1.9884x; 1.0504x over previous
import functools

import jax
import jax.numpy as jnp
from jax.experimental import pallas as pl
from jax.experimental.pallas import tpu as pltpu

BINS = 10
BLOCK_ROWS = 1024


def _g10(x, t):
    h = jnp.tanh(jnp.abs(x) * 0.5)
    d = 0.5 + 0.5 * h
    p = jnp.where(x >= 0.0, d, 1.0 - d)
    return jnp.abs(p - t) * BINS, d


def _hist_kernel(x_ref, t_ref, cnt_ref):
    i = pl.program_id(0)

    @pl.when(i == 0)
    def _init():
        for k in range(BINS):
            cnt_ref[0, k] = 0

    g10, _ = _g10(x_ref[...], t_ref[...])
    b = jnp.minimum(g10.astype(jnp.int32), BINS - 1)
    islo = b < 5
    sh6 = b * 6
    sh = jnp.where(islo, sh6, sh6 - 30)
    p = jnp.left_shift(jnp.int32(1), sh)
    plo = jnp.where(islo, p, 0)
    phi = p - plo
    for arr, base in ((plo, 0), (phi, 5)):
        s = arr
        for _ in range(5):
            h = s.shape[0] // 2
            s = s[:h] + s[h:]
        for f in range(5):
            cnt_ref[0, base + f] += jnp.sum((s >> (6 * f)) & 63)


def _loss_kernel(cnt_ref, x_ref, t_ref, out_ref, *, tot):
    del tot
    counts = [cnt_ref[0, k] for k in range(BINS)]
    n = functools.reduce(
        lambda a, b: a + b, [(c > 0).astype(jnp.int32) for c in counts]
    )
    nf = n.astype(jnp.float32)
    c = [1.0 / (jnp.maximum(cn, 1).astype(jnp.float32) * nf) for cn in counts]

    x = x_ref[...]
    t = t_ref[...]
    g10, d = _g10(x, t)
    w_lo = jnp.where(
        g10 >= 2.0,
        jnp.where(g10 >= 3.0, jnp.where(g10 >= 4.0, c[4], c[3]), c[2]),
        jnp.where(g10 >= 1.0, c[1], c[0]),
    )
    w_hi = jnp.where(
        g10 >= 7.0,
        jnp.where(g10 >= 8.0, jnp.where(g10 >= 9.0, c[9], c[8]), c[7]),
        jnp.where(g10 >= 6.0, c[6], c[5]),
    )
    w = jnp.where(g10 >= 5.0, w_hi, w_lo)
    bce = jnp.maximum(x, 0.0) - jnp.log(d) - x * t
    out_ref[...] = w * bce


def kernel(inputs, targets):
    rows, cols = inputs.shape
    tot = rows * cols
    grid = (rows // BLOCK_ROWS,)
    data_spec = pl.BlockSpec((BLOCK_ROWS, cols), lambda i: (i, 0))

    ge = pl.pallas_call(
        _hist_kernel,
        grid=grid,
        in_specs=[data_spec, data_spec],
        out_specs=pl.BlockSpec(memory_space=pltpu.SMEM),
        out_shape=jax.ShapeDtypeStruct((1, BINS), jnp.int32),
        compiler_params=pltpu.CompilerParams(
            dimension_semantics=("arbitrary",),
        ),
    )(inputs, targets)

    loss = pl.pallas_call(
        functools.partial(_loss_kernel, tot=tot),
        grid=grid,
        in_specs=[
            pl.BlockSpec(memory_space=pltpu.SMEM),
            data_spec,
            data_spec,
        ],
        out_specs=data_spec,
        out_shape=jax.ShapeDtypeStruct((rows, cols), jnp.float32),
        compiler_params=pltpu.CompilerParams(
            dimension_semantics=("parallel",),
        ),
    )(ge, inputs, targets)
    return loss


# speedup vs baseline: 1.9962x; 1.0039x over previous
import functools

import jax
import jax.numpy as jnp
from jax.experimental import pallas as pl
from jax.experimental.pallas import tpu as pltpu

BINS = 10
BLOCK_ROWS = 512


def _g10(x, t):
    h = jnp.tanh(jnp.abs(x) * 0.5)
    d = 0.5 + 0.5 * h
    p = jnp.where(x >= 0.0, d, 1.0 - d)
    return jnp.abs(p - t) * BINS, d


def _hist_kernel(x_ref, t_ref, cnt_ref):
    i = pl.program_id(0)

    @pl.when(i == 0)
    def _init():
        for k in range(BINS):
            cnt_ref[0, k] = 0

    g10, _ = _g10(x_ref[...], t_ref[...])
    b = jnp.minimum(g10.astype(jnp.int32), BINS - 1)
    islo = b < 5
    sh6 = b * 6
    sh = jnp.where(islo, sh6, sh6 - 30)
    p = jnp.left_shift(jnp.int32(1), sh)
    plo = jnp.where(islo, p, 0)
    phi = p - plo
    for arr, base in ((plo, 0), (phi, 5)):
        s = arr
        for _ in range(5):
            h = s.shape[0] // 2
            s = s[:h] + s[h:]
        for f in range(5):
            cnt_ref[0, base + f] += jnp.sum((s >> (6 * f)) & 63)


def _loss_kernel(cnt_ref, x_ref, t_ref, out_ref, *, tot):
    del tot
    counts = [cnt_ref[0, k] for k in range(BINS)]
    n = functools.reduce(
        lambda a, b: a + b, [(c > 0).astype(jnp.int32) for c in counts]
    )
    nf = n.astype(jnp.float32)
    c = [1.0 / (jnp.maximum(cn, 1).astype(jnp.float32) * nf) for cn in counts]

    x = x_ref[...]
    t = t_ref[...]
    g10, d = _g10(x, t)
    w_lo = jnp.where(
        g10 >= 2.0,
        jnp.where(g10 >= 3.0, jnp.where(g10 >= 4.0, c[4], c[3]), c[2]),
        jnp.where(g10 >= 1.0, c[1], c[0]),
    )
    w_hi = jnp.where(
        g10 >= 7.0,
        jnp.where(g10 >= 8.0, jnp.where(g10 >= 9.0, c[9], c[8]), c[7]),
        jnp.where(g10 >= 6.0, c[6], c[5]),
    )
    w = jnp.where(g10 >= 5.0, w_hi, w_lo)
    bce = jnp.maximum(x, 0.0) - jnp.log(d) - x * t
    out_ref[...] = w * bce


def kernel(inputs, targets):
    rows, cols = inputs.shape
    tot = rows * cols
    grid = (rows // BLOCK_ROWS,)
    data_spec = pl.BlockSpec((BLOCK_ROWS, cols), lambda i: (i, 0))

    ge = pl.pallas_call(
        _hist_kernel,
        grid=grid,
        in_specs=[data_spec, data_spec],
        out_specs=pl.BlockSpec(memory_space=pltpu.SMEM),
        out_shape=jax.ShapeDtypeStruct((1, BINS), jnp.int32),
        compiler_params=pltpu.CompilerParams(
            dimension_semantics=("arbitrary",),
        ),
    )(inputs, targets)

    loss = pl.pallas_call(
        functools.partial(_loss_kernel, tot=tot),
        grid=grid,
        in_specs=[
            pl.BlockSpec(memory_space=pltpu.SMEM),
            data_spec,
            data_spec,
        ],
        out_specs=data_spec,
        out_shape=jax.ShapeDtypeStruct((rows, cols), jnp.float32),
        compiler_params=pltpu.CompilerParams(
            dimension_semantics=("parallel",),
        ),
    )(ge, inputs, targets)
    return loss


# speedup vs baseline: 2.0078x; 1.0058x over previous
import functools

import jax
import jax.numpy as jnp
from jax.experimental import pallas as pl
from jax.experimental.pallas import tpu as pltpu

BINS = 10
BLOCK_ROWS = 256


def _g10(x, t):
    h = jnp.tanh(jnp.abs(x) * 0.5)
    d = 0.5 + 0.5 * h
    p = jnp.where(x >= 0.0, d, 1.0 - d)
    return jnp.abs(p - t) * BINS, d


def _hist_kernel(x_ref, t_ref, cnt_ref):
    i = pl.program_id(0)

    @pl.when(i == 0)
    def _init():
        for k in range(BINS):
            cnt_ref[0, k] = 0

    g10, _ = _g10(x_ref[...], t_ref[...])
    b = jnp.minimum(g10.astype(jnp.int32), BINS - 1)
    islo = b < 5
    sh6 = b * 6
    sh = jnp.where(islo, sh6, sh6 - 30)
    p = jnp.left_shift(jnp.int32(1), sh)
    plo = jnp.where(islo, p, 0)
    phi = p - plo
    for arr, base in ((plo, 0), (phi, 5)):
        s = arr
        for _ in range(5):
            h = s.shape[0] // 2
            s = s[:h] + s[h:]
        for f in range(5):
            cnt_ref[0, base + f] += jnp.sum((s >> (6 * f)) & 63)


def _loss_kernel(cnt_ref, x_ref, t_ref, out_ref, *, tot):
    del tot
    counts = [cnt_ref[0, k] for k in range(BINS)]
    n = functools.reduce(
        lambda a, b: a + b, [(c > 0).astype(jnp.int32) for c in counts]
    )
    nf = n.astype(jnp.float32)
    c = [1.0 / (jnp.maximum(cn, 1).astype(jnp.float32) * nf) for cn in counts]

    x = x_ref[...]
    t = t_ref[...]
    g10, d = _g10(x, t)
    w_lo = jnp.where(
        g10 >= 2.0,
        jnp.where(g10 >= 3.0, jnp.where(g10 >= 4.0, c[4], c[3]), c[2]),
        jnp.where(g10 >= 1.0, c[1], c[0]),
    )
    w_hi = jnp.where(
        g10 >= 7.0,
        jnp.where(g10 >= 8.0, jnp.where(g10 >= 9.0, c[9], c[8]), c[7]),
        jnp.where(g10 >= 6.0, c[6], c[5]),
    )
    w = jnp.where(g10 >= 5.0, w_hi, w_lo)
    bce = jnp.maximum(x, 0.0) - jnp.log(d) - x * t
    out_ref[...] = w * bce


def kernel(inputs, targets):
    rows, cols = inputs.shape
    tot = rows * cols
    grid = (rows // BLOCK_ROWS,)
    data_spec = pl.BlockSpec((BLOCK_ROWS, cols), lambda i: (i, 0))

    ge = pl.pallas_call(
        _hist_kernel,
        grid=grid,
        in_specs=[data_spec, data_spec],
        out_specs=pl.BlockSpec(memory_space=pltpu.SMEM),
        out_shape=jax.ShapeDtypeStruct((1, BINS), jnp.int32),
        compiler_params=pltpu.CompilerParams(
            dimension_semantics=("arbitrary",),
        ),
    )(inputs, targets)

    loss = pl.pallas_call(
        functools.partial(_loss_kernel, tot=tot),
        grid=grid,
        in_specs=[
            pl.BlockSpec(memory_space=pltpu.SMEM),
            data_spec,
            data_spec,
        ],
        out_specs=data_spec,
        out_shape=jax.ShapeDtypeStruct((rows, cols), jnp.float32),
        compiler_params=pltpu.CompilerParams(
            dimension_semantics=("parallel",),
        ),
    )(ge, inputs, targets)
    return loss
